# baseline (speedup 1.0000x reference)
import jax
import jax.numpy as jnp
from jax.experimental import pallas as pl

DIM = 1024
HEADS = 16
BUCKET = 64
S = 2048
HD = DIM // HEADS
QBLK = 256
NQB = S // QBLK


def _proj_hash_kernel(xq_ref, xk_ref, xv_ref, wq_ref, bq_ref, wk_ref, bk_ref,
                      wv_ref, bv_ref, lsh_ref,
                      Q_ref, K_ref, V_ref, qh_ref, kh_ref, vsum_ref):
    lsh = lsh_ref[...]
    q = jnp.dot(xq_ref[...], wq_ref[...],
                preferred_element_type=jnp.float32) + bq_ref[...]
    Q_ref[...] = q.astype(jnp.bfloat16)
    qh_ref[...] = jnp.argmax(jnp.dot(q, lsh, preferred_element_type=jnp.float32),
                             axis=-1).astype(jnp.int32).reshape(1, QBLK)
    k = jnp.dot(xk_ref[...], wk_ref[...],
                preferred_element_type=jnp.float32) + bk_ref[...]
    K_ref[...] = k.astype(jnp.bfloat16)
    kh_ref[...] = jnp.argmax(jnp.dot(k, lsh, preferred_element_type=jnp.float32),
                             axis=-1).astype(jnp.int32).reshape(1, QBLK)
    v = jnp.dot(xv_ref[...], wv_ref[...],
                preferred_element_type=jnp.float32) + bv_ref[...]
    V_ref[...] = v.astype(jnp.bfloat16)

    i = pl.program_id(0)

    @pl.when(i == 0)
    def _():
        vsum_ref[...] = jnp.zeros((1, DIM), jnp.float32)

    vsum_ref[...] += jnp.sum(v, axis=0, keepdims=True)


def _attn_kernel(qh_ref, kh_ref, Q_ref, K_ref, V_ref, vsum_ref, o_ref):
    qi = pl.program_id(0)
    qh = qh_ref[0, pl.ds(qi * QBLK, QBLK)]
    kh = kh_ref[0, :]
    qhb = jax.lax.broadcast_in_dim(qh, (QBLK, S), (0,))
    khb = jax.lax.broadcast_in_dim(kh, (QBLK, S), (1,))
    mask = qhb == khb
    maskadd = jnp.where(mask, 0.0, -1e9)
    has = jnp.max(mask.astype(jnp.float32), axis=-1, keepdims=True) > 0.0
    vs = vsum_ref[...]
    qblk = Q_ref[...] * jnp.bfloat16(0.125)
    for h in range(HEADS):
        sl = slice(h * HD, (h + 1) * HD)
        s = jax.lax.dot_general(qblk[:, sl], K_ref[:, sl],
                                (((1,), (1,)), ((), ())),
                                preferred_element_type=jnp.float32) + maskadd
        m = jnp.max(s, axis=-1, keepdims=True)
        e = jnp.exp(s - m)
        o = jnp.dot(e.astype(jnp.bfloat16), V_ref[:, sl],
                    preferred_element_type=jnp.float32)
        o = o / jnp.sum(e, axis=-1, keepdims=True)
        o_ref[:, sl] = jnp.where(
            jax.lax.broadcast_in_dim(has[:, 0], (QBLK, HD), (0,)),
            o, jax.lax.broadcast_in_dim(vs[0, sl] * (1.0 / S),
                                        (QBLK, HD), (1,))).astype(jnp.bfloat16)


def _rank_kernel(qh_ref, rank_ref):
    h = qh_ref[0, :]
    hb = jax.lax.broadcast_in_dim(h, (S, BUCKET), (0,))
    bid = jax.lax.broadcasted_iota(jnp.int32, (S, BUCKET), 1)
    oh = (hb == bid).astype(jnp.float32)
    incl = oh
    shift = 1
    while shift < S:
        incl = incl + jnp.concatenate(
            [jnp.zeros((shift, BUCKET), jnp.float32), incl[:S - shift]], axis=0)
        shift *= 2
    counts = incl[S - 1:S, :]
    cs = counts
    shift = 1
    while shift < BUCKET:
        cs = cs + jnp.concatenate(
            [jnp.zeros((1, shift), jnp.float32), cs[:, :BUCKET - shift]], axis=1)
        shift *= 2
    offsets = cs - counts
    rank_f = jnp.sum(oh * (incl - 1.0 + offsets), axis=1)
    rank_ref[...] = rank_f.astype(jnp.int32).reshape(1, S)


def _gather_proj_kernel(rank_ref, attn_ref, wo_ref, bo_ref, out_ref):
    i = pl.program_id(0)
    rows = jax.lax.broadcasted_iota(jnp.int32, (QBLK, S), 0) + i * QBLK
    rk = jax.lax.broadcast_in_dim(rank_ref[0, :], (QBLK, S), (1,))
    m2 = (rows == rk).astype(jnp.bfloat16)
    g = jnp.dot(m2, attn_ref[...], preferred_element_type=jnp.float32)
    out_ref[...] = jnp.dot(g, wo_ref[...], preferred_element_type=jnp.float32) + bo_ref[...]


def kernel(query, key, value, Wq, bq, Wk, bk, Wv, bv, Wo, bo, lsh_proj):
    xq, xk, xv = query[0], key[0], value[0]
    bq2, bk2, bv2, bo2 = (b.reshape(1, DIM) for b in (bq, bk, bv, bo))

    full = lambda shape: pl.BlockSpec(shape, lambda i: (0, 0))
    rowblk = pl.BlockSpec((QBLK, DIM), lambda i: (i, 0))
    hashblk = pl.BlockSpec((1, QBLK), lambda i: (0, i))

    Q, K, V, qh, kh, vsum = pl.pallas_call(
        _proj_hash_kernel,
        grid=(NQB,),
        in_specs=[rowblk, rowblk, rowblk,
                  full((DIM, DIM)), full((1, DIM)),
                  full((DIM, DIM)), full((1, DIM)),
                  full((DIM, DIM)), full((1, DIM)),
                  full((DIM, BUCKET))],
        out_specs=[rowblk, rowblk, rowblk, hashblk, hashblk, full((1, DIM))],
        out_shape=[jax.ShapeDtypeStruct((S, DIM), jnp.bfloat16),
                   jax.ShapeDtypeStruct((S, DIM), jnp.bfloat16),
                   jax.ShapeDtypeStruct((S, DIM), jnp.bfloat16),
                   jax.ShapeDtypeStruct((1, S), jnp.int32),
                   jax.ShapeDtypeStruct((1, S), jnp.int32),
                   jax.ShapeDtypeStruct((1, DIM), jnp.float32)],
    )(xq, xk, xv, Wq, bq2, Wk, bk2, Wv, bv2, lsh_proj)

    attn = pl.pallas_call(
        _attn_kernel,
        grid=(NQB,),
        in_specs=[pl.BlockSpec((1, S), lambda qi: (0, 0)),
                  pl.BlockSpec((1, S), lambda qi: (0, 0)),
                  pl.BlockSpec((QBLK, DIM), lambda qi: (qi, 0)),
                  pl.BlockSpec((S, DIM), lambda qi: (0, 0)),
                  pl.BlockSpec((S, DIM), lambda qi: (0, 0)),
                  pl.BlockSpec((1, DIM), lambda qi: (0, 0))],
        out_specs=pl.BlockSpec((QBLK, DIM), lambda qi: (qi, 0)),
        out_shape=jax.ShapeDtypeStruct((S, DIM), jnp.bfloat16),
    )(qh, kh, Q, K, V, vsum)

    rank = pl.pallas_call(
        _rank_kernel,
        grid=(1,),
        in_specs=[pl.BlockSpec((1, S), lambda i: (0, 0))],
        out_specs=pl.BlockSpec((1, S), lambda i: (0, 0)),
        out_shape=jax.ShapeDtypeStruct((1, S), jnp.int32),
    )(qh)

    out = pl.pallas_call(
        _gather_proj_kernel,
        grid=(NQB,),
        in_specs=[pl.BlockSpec((1, S), lambda i: (0, 0)),
                  pl.BlockSpec((S, DIM), lambda i: (0, 0)),
                  pl.BlockSpec((DIM, DIM), lambda i: (0, 0)),
                  pl.BlockSpec((1, DIM), lambda i: (0, 0))],
        out_specs=rowblk,
        out_shape=jax.ShapeDtypeStruct((S, DIM), jnp.float32),
    )(rank, attn, Wo, bo2)

    return out.reshape(1, S, DIM)


# speedup vs baseline: 1.0425x; 1.0425x over previous
import jax
import jax.numpy as jnp
from jax.experimental import pallas as pl

DIM = 1024
HEADS = 16
BUCKET = 64
S = 2048
HD = DIM // HEADS
QBLK = 256
NQB = S // QBLK


def _proj_hash_kernel(xq_ref, xk_ref, xv_ref, wq_ref, bq_ref, wk_ref, bk_ref,
                      wv_ref, bv_ref, lsh_ref,
                      Q_ref, K_ref, V_ref, qh_ref, kh_ref, vsum_ref):
    lsh = lsh_ref[...]
    q = jnp.dot(xq_ref[...], wq_ref[...],
                preferred_element_type=jnp.float32) + bq_ref[...]
    Q_ref[...] = q
    qh_ref[...] = jnp.argmax(jnp.dot(q, lsh, preferred_element_type=jnp.float32),
                             axis=-1).astype(jnp.int32).reshape(1, QBLK)
    k = jnp.dot(xk_ref[...], wk_ref[...],
                preferred_element_type=jnp.float32) + bk_ref[...]
    K_ref[...] = k
    kh_ref[...] = jnp.argmax(jnp.dot(k, lsh, preferred_element_type=jnp.float32),
                             axis=-1).astype(jnp.int32).reshape(1, QBLK)
    v = jnp.dot(xv_ref[...], wv_ref[...],
                preferred_element_type=jnp.float32) + bv_ref[...]
    V_ref[...] = v

    i = pl.program_id(0)

    @pl.when(i == 0)
    def _():
        vsum_ref[...] = jnp.zeros((1, DIM), jnp.float32)

    vsum_ref[...] += jnp.sum(v, axis=0, keepdims=True)


def _attn_kernel(qh_ref, kh_ref, Q_ref, K_ref, V_ref, vsum_ref, o_ref):
    qi = pl.program_id(0)
    qh = qh_ref[0, pl.ds(qi * QBLK, QBLK)]
    kh = kh_ref[0, :]
    qhb = jax.lax.broadcast_in_dim(qh, (QBLK, S), (0,))
    khb = jax.lax.broadcast_in_dim(kh, (QBLK, S), (1,))
    mask = qhb == khb
    maskadd = jnp.where(mask, 0.0, -1e9)
    has = jnp.max(mask.astype(jnp.float32), axis=-1, keepdims=True) > 0.0
    vs = vsum_ref[...]
    qblk = Q_ref[...] * 0.125
    for h in range(HEADS):
        sl = slice(h * HD, (h + 1) * HD)
        s = jax.lax.dot_general(qblk[:, sl], K_ref[:, sl],
                                (((1,), (1,)), ((), ())),
                                preferred_element_type=jnp.float32) + maskadd
        m = jnp.max(s, axis=-1, keepdims=True)
        e = jnp.exp(s - m)
        o = jnp.dot(e, V_ref[:, sl], preferred_element_type=jnp.float32)
        o = o / jnp.sum(e, axis=-1, keepdims=True)
        o_ref[:, sl] = jnp.where(
            jax.lax.broadcast_in_dim(has[:, 0], (QBLK, HD), (0,)),
            o, jax.lax.broadcast_in_dim(vs[0, sl] * (1.0 / S), (QBLK, HD), (1,)))


def _rank_kernel(qh_ref, rank_ref):
    h = qh_ref[0, :]
    hb = jax.lax.broadcast_in_dim(h, (S, BUCKET), (0,))
    bid = jax.lax.broadcasted_iota(jnp.int32, (S, BUCKET), 1)
    oh = (hb == bid).astype(jnp.float32)
    incl = oh
    shift = 1
    while shift < S:
        incl = incl + jnp.concatenate(
            [jnp.zeros((shift, BUCKET), jnp.float32), incl[:S - shift]], axis=0)
        shift *= 2
    counts = incl[S - 1:S, :]
    cs = counts
    shift = 1
    while shift < BUCKET:
        cs = cs + jnp.concatenate(
            [jnp.zeros((1, shift), jnp.float32), cs[:, :BUCKET - shift]], axis=1)
        shift *= 2
    offsets = cs - counts
    rank_f = jnp.sum(oh * (incl - 1.0 + offsets), axis=1)
    rank_ref[...] = rank_f.astype(jnp.int32).reshape(1, S)


def _gather_proj_kernel(rank_ref, attn_ref, wo_ref, bo_ref, out_ref):
    i = pl.program_id(0)
    rows = jax.lax.broadcasted_iota(jnp.int32, (QBLK, S), 0) + i * QBLK
    rk = jax.lax.broadcast_in_dim(rank_ref[0, :], (QBLK, S), (1,))
    m2 = (rows == rk).astype(jnp.float32)
    g = jnp.dot(m2, attn_ref[...], preferred_element_type=jnp.float32)
    out_ref[...] = jnp.dot(g, wo_ref[...], preferred_element_type=jnp.float32) + bo_ref[...]


def kernel(query, key, value, Wq, bq, Wk, bk, Wv, bv, Wo, bo, lsh_proj):
    xq, xk, xv = query[0], key[0], value[0]
    bq2, bk2, bv2, bo2 = (b.reshape(1, DIM) for b in (bq, bk, bv, bo))

    full = lambda shape: pl.BlockSpec(shape, lambda i: (0, 0))
    rowblk = pl.BlockSpec((QBLK, DIM), lambda i: (i, 0))
    hashblk = pl.BlockSpec((1, QBLK), lambda i: (0, i))

    Q, K, V, qh, kh, vsum = pl.pallas_call(
        _proj_hash_kernel,
        grid=(NQB,),
        in_specs=[rowblk, rowblk, rowblk,
                  full((DIM, DIM)), full((1, DIM)),
                  full((DIM, DIM)), full((1, DIM)),
                  full((DIM, DIM)), full((1, DIM)),
                  full((DIM, BUCKET))],
        out_specs=[rowblk, rowblk, rowblk, hashblk, hashblk, full((1, DIM))],
        out_shape=[jax.ShapeDtypeStruct((S, DIM), jnp.float32),
                   jax.ShapeDtypeStruct((S, DIM), jnp.float32),
                   jax.ShapeDtypeStruct((S, DIM), jnp.float32),
                   jax.ShapeDtypeStruct((1, S), jnp.int32),
                   jax.ShapeDtypeStruct((1, S), jnp.int32),
                   jax.ShapeDtypeStruct((1, DIM), jnp.float32)],
    )(xq, xk, xv, Wq, bq2, Wk, bk2, Wv, bv2, lsh_proj)

    attn = pl.pallas_call(
        _attn_kernel,
        grid=(NQB,),
        in_specs=[pl.BlockSpec((1, S), lambda qi: (0, 0)),
                  pl.BlockSpec((1, S), lambda qi: (0, 0)),
                  pl.BlockSpec((QBLK, DIM), lambda qi: (qi, 0)),
                  pl.BlockSpec((S, DIM), lambda qi: (0, 0)),
                  pl.BlockSpec((S, DIM), lambda qi: (0, 0)),
                  pl.BlockSpec((1, DIM), lambda qi: (0, 0))],
        out_specs=pl.BlockSpec((QBLK, DIM), lambda qi: (qi, 0)),
        out_shape=jax.ShapeDtypeStruct((S, DIM), jnp.float32),
    )(qh, kh, Q, K, V, vsum)

    rank = pl.pallas_call(
        _rank_kernel,
        grid=(1,),
        in_specs=[pl.BlockSpec((1, S), lambda i: (0, 0))],
        out_specs=pl.BlockSpec((1, S), lambda i: (0, 0)),
        out_shape=jax.ShapeDtypeStruct((1, S), jnp.int32),
    )(qh)

    out = pl.pallas_call(
        _gather_proj_kernel,
        grid=(NQB,),
        in_specs=[pl.BlockSpec((1, S), lambda i: (0, 0)),
                  pl.BlockSpec((S, DIM), lambda i: (0, 0)),
                  pl.BlockSpec((DIM, DIM), lambda i: (0, 0)),
                  pl.BlockSpec((1, DIM), lambda i: (0, 0))],
        out_specs=rowblk,
        out_shape=jax.ShapeDtypeStruct((S, DIM), jnp.float32),
    )(rank, attn, Wo, bo2)

    return out.reshape(1, S, DIM)
